# baseline (speedup 1.0000x reference)
import jax
import jax.numpy as jnp
from jax import lax
from jax.experimental import pallas as pl
from jax.experimental.pallas import tpu as pltpu
from jax.experimental.pallas import tpu_sc as plsc

N_EDGES = 3_200_000
N_NODES = 100_000
D_EDGE = 16
D_FEAT = 128

TC_GRID = 100
TC_EBLK = 11_520
TC_ELANES = TC_GRID * TC_EBLK
EACC_W = 1_280

NW = 32
CHUNK = 3200
SC_BASE = TC_ELANES // CHUNK
PER_W = (N_EDGES - TC_ELANES) // (NW * CHUNK)
MAIN = PER_W - 1

NBLK = N_NODES // TC_GRID


def _sc_body(edges_hbm, pe_hbm, buf0, buf1, pbuf, sem0, sem1):
    wid = lax.axis_index("s") * 2 + lax.axis_index("c")
    base = SC_BASE + wid * PER_W

    def start(ci, buf, sem):
        pltpu.async_copy(edges_hbm.at[:, pl.ds((base + ci) * CHUNK, CHUNK)],
                         buf, sem)

    def drain(buf, sem):
        pltpu.make_async_copy(edges_hbm.at[:, pl.ds(0, CHUNK)], buf,
                              sem).wait()

    def accum(buf, acc):
        def body(k, a):
            for u in range(2):
                a = tuple(
                    a[r] + buf[r, pl.ds((2 * k + u) * 16, 16)]
                    for r in range(16)
                )
            return a
        return lax.fori_loop(0, CHUNK // 32, body, acc)

    start(0, buf0, sem0)

    def pair_body(i, acc):
        start(2 * i + 1, buf1, sem1)
        drain(buf0, sem0)
        acc = accum(buf0, acc)
        start(2 * i + 2, buf0, sem0)
        drain(buf1, sem1)
        return accum(buf1, acc)

    zero = tuple(jnp.zeros((16,), jnp.float32) for _ in range(16))
    acc = lax.fori_loop(0, MAIN // 2, pair_body, zero)
    start(PER_W - 1, buf1, sem1)
    drain(buf0, sem0)
    acc = accum(buf0, acc)
    drain(buf1, sem1)
    acc = accum(buf1, acc)

    for r in range(16):
        pbuf[r, :] = acc[r]
    pltpu.sync_copy(pbuf, pe_hbm.at[pl.ds(wid * 16, 16)])


def _sc_edge_sums(edges_t):
    mesh = plsc.VectorSubcoreMesh(core_axis_name="c", subcore_axis_name="s")
    return pl.kernel(
        _sc_body,
        mesh=mesh,
        out_type=jax.ShapeDtypeStruct((NW * 16, D_EDGE), jnp.float32),
        scratch_types=[
            pltpu.VMEM((D_EDGE, CHUNK), jnp.float32),
            pltpu.VMEM((D_EDGE, CHUNK), jnp.float32),
            pltpu.VMEM((16, D_EDGE), jnp.float32),
            pltpu.SemaphoreType.DMA,
            pltpu.SemaphoreType.DMA,
        ],
    )(edges_t)


def _tc_main_body(nodes_ref, edges_ref, nsum_ref, esum_ref, nacc, eacc):
    g = pl.program_id(0)

    @pl.when(g == 0)
    def _init():
        nacc[...] = jnp.zeros_like(nacc)
        eacc[...] = jnp.zeros_like(eacc)

    nacc[...] += jnp.sum(nodes_ref[...], axis=0, keepdims=True)
    e = eacc[...]
    for s in range(TC_EBLK // EACC_W):
        e = e + edges_ref[:, pl.ds(s * EACC_W, EACC_W)]
    eacc[...] = e

    @pl.when(g == TC_GRID - 1)
    def _fin():
        nsum_ref[...] = nacc[...]
        esum_ref[...] = eacc[...]


def _tc_main(nodes_data, edges_t):
    return pl.pallas_call(
        _tc_main_body,
        grid=(TC_GRID,),
        in_specs=[
            pl.BlockSpec((NBLK, 128), lambda g: (g, 0)),
            pl.BlockSpec((D_EDGE, TC_EBLK), lambda g: (0, g)),
        ],
        out_specs=[
            pl.BlockSpec((1, 128), lambda g: (0, 0)),
            pl.BlockSpec((D_EDGE, EACC_W), lambda g: (0, 0)),
        ],
        out_shape=[
            jax.ShapeDtypeStruct((1, 128), jnp.float32),
            jax.ShapeDtypeStruct((D_EDGE, EACC_W), jnp.float32),
        ],
        scratch_shapes=[
            pltpu.VMEM((1, 128), jnp.float32),
            pltpu.VMEM((D_EDGE, EACC_W), jnp.float32),
        ],
    )(nodes_data, edges_t)


def _tc_fin_body(glob_ref, pe_ref, nsum_ref, esum_ref, WgT_ref, WeRep_ref,
                 WeT_ref, WnT_ref, b_ref, out_ref):
    rowsum = jnp.dot(pe_ref[...], jnp.ones((16, 1), jnp.float32),
                     preferred_element_type=jnp.float32)
    e_sc = lax.dot_general(rowsum, WeRep_ref[...], (((0,), (0,)), ((), ())),
                           preferred_element_type=jnp.float32)
    erow = jnp.dot(esum_ref[...], jnp.ones((EACC_W, 1), jnp.float32),
                   preferred_element_type=jnp.float32)
    e_tc = lax.dot_general(erow, WeT_ref[...], (((0,), (0,)), ((), ())),
                           preferred_element_type=jnp.float32)
    n_row = nsum_ref[...] * (1.0 / N_NODES)
    out_ref[...] = (
        jnp.dot(glob_ref[...], WgT_ref[...],
                preferred_element_type=jnp.float32)
        + (e_sc + e_tc) * (1.0 / N_EDGES)
        + jnp.dot(n_row, WnT_ref[...], preferred_element_type=jnp.float32)
        + b_ref[...])


def kernel(global_data, nodes_data, edges_data, W, b):
    edges_t = edges_data.T
    pe = _sc_edge_sums(edges_t)
    nsum, esum = _tc_main(nodes_data, edges_t)
    WT = W.T
    WeT = WT[128:144]
    WeRep = jnp.tile(WeT, (NW, 1))
    out = pl.pallas_call(
        _tc_fin_body,
        out_shape=jax.ShapeDtypeStruct((1, 128), jnp.float32),
    )(global_data[None, :], pe, nsum, esum, WT[:128], WeRep, WeT, WT[144:],
      b[None, :])
    return out[0]


# speedup vs baseline: 1.1076x; 1.1076x over previous
import jax
import jax.numpy as jnp
from jax import lax
from jax.experimental import pallas as pl
from jax.experimental.pallas import tpu as pltpu

N_EDGES = 3_200_000
N_NODES = 100_000
D_EDGE = 16

GRID = 100
EBLK = N_EDGES // GRID
EACC_W = 1280
NBLK = N_NODES // GRID


def _body(glob_ref, nodes_ref, edges_ref, WgT_ref, WeT_ref, WnT_ref, b_ref,
          out_ref, nacc, eacc):
    g = pl.program_id(0)

    @pl.when(g == 0)
    def _init():
        nacc[...] = jnp.zeros_like(nacc)
        eacc[...] = jnp.zeros_like(eacc)

    nacc[...] += jnp.sum(nodes_ref[...], axis=0, keepdims=True)
    e = eacc[...]
    for s in range(EBLK // EACC_W):
        e = e + edges_ref[:, pl.ds(s * EACC_W, EACC_W)]
    eacc[...] = e

    @pl.when(g == GRID - 1)
    def _fin():
        erow = jnp.dot(eacc[...], jnp.ones((EACC_W, 1), jnp.float32),
                       preferred_element_type=jnp.float32)
        e_out = lax.dot_general(
            erow, WeT_ref[...], (((0,), (0,)), ((), ())),
            preferred_element_type=jnp.float32)
        n_row = nacc[...] * (1.0 / N_NODES)
        out_ref[...] = (
            jnp.dot(glob_ref[...], WgT_ref[...],
                    preferred_element_type=jnp.float32)
            + e_out * (1.0 / N_EDGES)
            + jnp.dot(n_row, WnT_ref[...], preferred_element_type=jnp.float32)
            + b_ref[...])


def kernel(global_data, nodes_data, edges_data, W, b):
    edges_t = edges_data.T
    WT = W.T
    out = pl.pallas_call(
        _body,
        grid=(GRID,),
        in_specs=[
            pl.BlockSpec((1, 128), lambda g: (0, 0)),
            pl.BlockSpec((NBLK, 128), lambda g: (g, 0)),
            pl.BlockSpec((D_EDGE, EBLK), lambda g: (0, g)),
            pl.BlockSpec((128, 128), lambda g: (0, 0)),
            pl.BlockSpec((16, 128), lambda g: (0, 0)),
            pl.BlockSpec((128, 128), lambda g: (0, 0)),
            pl.BlockSpec((1, 128), lambda g: (0, 0)),
        ],
        out_specs=pl.BlockSpec((1, 128), lambda g: (0, 0)),
        out_shape=jax.ShapeDtypeStruct((1, 128), jnp.float32),
        scratch_shapes=[
            pltpu.VMEM((1, 128), jnp.float32),
            pltpu.VMEM((D_EDGE, EACC_W), jnp.float32),
        ],
    )(global_data[None, :], nodes_data, edges_t, WT[:128], WT[128:144],
      WT[144:], b[None, :])
    return out[0]


# speedup vs baseline: 1.4690x; 1.3264x over previous
import jax
import jax.numpy as jnp
from jax import lax
from jax.experimental import pallas as pl
from jax.experimental.pallas import tpu as pltpu

N_EDGES = 3_200_000
N_NODES = 100_000
D_EDGE = 16

GRID = 25
EBLK = N_EDGES // GRID
EACC_W = 3200
NBLK = N_NODES // GRID


def _body(glob_ref, nodes_ref, edges_ref, WgT_ref, WeT_ref, WnT_ref, b_ref,
          out_ref, nacc, eacc):
    g = pl.program_id(0)

    @pl.when(g == 0)
    def _init():
        nacc[...] = jnp.zeros_like(nacc)
        eacc[...] = jnp.zeros_like(eacc)

    nacc[...] += jnp.sum(nodes_ref[...], axis=0, keepdims=True)
    e = eacc[...]
    for s in range(EBLK // EACC_W):
        e = e + edges_ref[:, pl.ds(s * EACC_W, EACC_W)]
    eacc[...] = e

    @pl.when(g == GRID - 1)
    def _fin():
        erow = jnp.dot(eacc[...], jnp.ones((EACC_W, 1), jnp.float32),
                       preferred_element_type=jnp.float32)
        e_out = lax.dot_general(
            erow, WeT_ref[...], (((0,), (0,)), ((), ())),
            preferred_element_type=jnp.float32)
        n_row = nacc[...] * (1.0 / N_NODES)
        out_ref[...] = (
            jnp.dot(glob_ref[...], WgT_ref[...],
                    preferred_element_type=jnp.float32)
            + e_out * (1.0 / N_EDGES)
            + jnp.dot(n_row, WnT_ref[...], preferred_element_type=jnp.float32)
            + b_ref[...])


def kernel(global_data, nodes_data, edges_data, W, b):
    edges_t = edges_data.T
    WT = W.T
    out = pl.pallas_call(
        _body,
        grid=(GRID,),
        in_specs=[
            pl.BlockSpec((1, 128), lambda g: (0, 0)),
            pl.BlockSpec((NBLK, 128), lambda g: (g, 0)),
            pl.BlockSpec((D_EDGE, EBLK), lambda g: (0, g)),
            pl.BlockSpec((128, 128), lambda g: (0, 0)),
            pl.BlockSpec((16, 128), lambda g: (0, 0)),
            pl.BlockSpec((128, 128), lambda g: (0, 0)),
            pl.BlockSpec((1, 128), lambda g: (0, 0)),
        ],
        out_specs=pl.BlockSpec((1, 128), lambda g: (0, 0)),
        out_shape=jax.ShapeDtypeStruct((1, 128), jnp.float32),
        scratch_shapes=[
            pltpu.VMEM((1, 128), jnp.float32),
            pltpu.VMEM((D_EDGE, EACC_W), jnp.float32),
        ],
    )(global_data[None, :], nodes_data, edges_t, WT[:128], WT[128:144],
      WT[144:], b[None, :])
    return out[0]
